# Initial kernel scaffold; baseline (speedup 1.0000x reference)
import jax
import jax.numpy as jnp
from jax.experimental import pallas as pl


def kernel(x, embedding):
    raise NotImplementedError("write your pallas kernel here")


# speedup vs baseline: 1.2368x; 1.2368x over previous
import functools

import jax
import jax.numpy as jnp
from jax.experimental import pallas as pl
from jax.experimental.pallas import tpu as pltpu

_NUM_EMB = 1024
_DIM = 64
_ROWS = 18432
_TILE = 512
_GRID = _ROWS // _TILE
_TOTAL = _ROWS * _DIM
_CCOST = 0.25


def _vq_body(x_ref, e_ref, loss_ref, q_ref, perp_ref, idx_ref, lacc, hist):
    i = pl.program_id(0)

    @pl.when(i == 0)
    def _init():
        lacc[0, 0] = 0.0
        hist[...] = jnp.zeros_like(hist)

    xt = x_ref[...]
    emb = e_ref[...]
    s1 = jnp.sum(xt * xt, axis=1, keepdims=True)
    s2 = jnp.sum(emb * emb, axis=0, keepdims=True)
    mm = jnp.dot(xt, emb, preferred_element_type=jnp.float32)
    dist = s1 - 2.0 * mm + s2

    dmin = jnp.min(dist, axis=1, keepdims=True)
    iota = jax.lax.broadcasted_iota(jnp.int32, (_TILE, _NUM_EMB), 1)
    idx = jnp.min(jnp.where(dist == dmin, iota, _NUM_EMB), axis=1)
    idx_ref[0, 0, :] = idx

    onehot = (iota == idx[:, None]).astype(jnp.float32)
    q = jax.lax.dot_general(
        onehot, emb, (((1,), (1,)), ((), ())),
        preferred_element_type=jnp.float32)
    diff = q - xt
    q_ref[...] = xt + diff
    lacc[0, 0] += jnp.sum(diff * diff)
    hist[...] += jnp.sum(onehot, axis=0, keepdims=True)

    @pl.when(i == _GRID - 1)
    def _fini():
        m = lacc[0, 0] / _TOTAL
        loss_ref[...] = jnp.full((1, 1), m + _CCOST * m, jnp.float32)
        avg = hist[...] / _ROWS
        ent = -jnp.sum(avg * jnp.log(avg + 1e-10))
        perp_ref[...] = jnp.full((1, 1), jnp.exp(ent), jnp.float32)


@functools.partial(jax.jit, static_argnames=("interpret",))
def kernel(x, embedding, interpret=False):
    flat_x = x.reshape(_ROWS, _DIM)
    loss, qst, perp, idx = pl.pallas_call(
        _vq_body,
        grid=(_GRID,),
        in_specs=[
            pl.BlockSpec((_TILE, _DIM), lambda i: (i, 0)),
            pl.BlockSpec((_DIM, _NUM_EMB), lambda i: (0, 0)),
        ],
        out_specs=[
            pl.BlockSpec((1, 1), lambda i: (0, 0)),
            pl.BlockSpec((_TILE, _DIM), lambda i: (i, 0)),
            pl.BlockSpec((1, 1), lambda i: (0, 0)),
            pl.BlockSpec((1, 1, _TILE), lambda i: (i, 0, 0)),
        ],
        out_shape=[
            jax.ShapeDtypeStruct((1, 1), jnp.float32),
            jax.ShapeDtypeStruct((_ROWS, _DIM), jnp.float32),
            jax.ShapeDtypeStruct((1, 1), jnp.float32),
            jax.ShapeDtypeStruct((_GRID, 1, _TILE), jnp.int32),
        ],
        scratch_shapes=[
            pltpu.SMEM((1, 1), jnp.float32),
            pltpu.VMEM((1, _NUM_EMB), jnp.float32),
        ],
        compiler_params=pltpu.CompilerParams(
            dimension_semantics=("arbitrary",)),
        interpret=interpret,
    )(flat_x, embedding)
    quantized_st = qst.reshape(x.shape)
    encoding_indices = idx.reshape(x.shape[:1] + x.shape[2:])
    return (loss.reshape(()), quantized_st, perp.reshape(()), encoding_indices)


# speedup vs baseline: 1.3806x; 1.1163x over previous
import functools

import jax
import jax.numpy as jnp
from jax.experimental import pallas as pl
from jax.experimental.pallas import tpu as pltpu

_NUM_EMB = 1024
_DIM = 64
_ROWS = 18432
_TILE = 2048
_GRID = _ROWS // _TILE
_TOTAL = _ROWS * _DIM
_CCOST = 0.25


def _vq_body(x_ref, e_ref, loss_ref, q_ref, perp_ref, idx_ref, lacc, hist, s2t):
    i = pl.program_id(0)

    xt = x_ref[...]
    emb = e_ref[...]

    @pl.when(i == 0)
    def _init():
        lacc[0, 0] = 0.0
        hist[...] = jnp.zeros_like(hist)
        s2t[...] = jnp.sum(emb * emb, axis=0, keepdims=True)

    s1 = jnp.sum(xt * xt, axis=1, keepdims=True)
    mm = jnp.dot(xt, emb, preferred_element_type=jnp.float32)
    dist = s1 - 2.0 * mm + s2t[...]

    dmin = jnp.min(dist, axis=1, keepdims=True)
    iotaf = jax.lax.broadcasted_iota(
        jnp.int32, (_TILE, _NUM_EMB), 1).astype(jnp.float32)
    idxf = jnp.min(jnp.where(dist == dmin, iotaf, float(_NUM_EMB)),
                   axis=1, keepdims=True)
    idx_ref[0, 0, :] = idxf.astype(jnp.int32).reshape(_TILE)

    onehot = (iotaf == idxf).astype(jnp.float32)
    q = jax.lax.dot_general(
        onehot, emb, (((1,), (1,)), ((), ())),
        preferred_element_type=jnp.float32)
    diff = q - xt
    q_ref[...] = xt + diff
    lacc[0, 0] += jnp.sum(diff * diff)
    hist[...] += jnp.sum(onehot, axis=0, keepdims=True)

    @pl.when(i == _GRID - 1)
    def _fini():
        m = lacc[0, 0] / _TOTAL
        loss_ref[...] = jnp.full((1, 1), m + _CCOST * m, jnp.float32)
        avg = hist[...] / _ROWS
        ent = -jnp.sum(avg * jnp.log(avg + 1e-10))
        perp_ref[...] = jnp.full((1, 1), jnp.exp(ent), jnp.float32)


@functools.partial(jax.jit, static_argnames=("interpret",))
def kernel(x, embedding, interpret=False):
    flat_x = x.reshape(_ROWS, _DIM)
    loss, qst, perp, idx = pl.pallas_call(
        _vq_body,
        grid=(_GRID,),
        in_specs=[
            pl.BlockSpec((_TILE, _DIM), lambda i: (i, 0)),
            pl.BlockSpec((_DIM, _NUM_EMB), lambda i: (0, 0)),
        ],
        out_specs=[
            pl.BlockSpec((1, 1), lambda i: (0, 0)),
            pl.BlockSpec((_TILE, _DIM), lambda i: (i, 0)),
            pl.BlockSpec((1, 1), lambda i: (0, 0)),
            pl.BlockSpec((1, 1, _TILE), lambda i: (i, 0, 0)),
        ],
        out_shape=[
            jax.ShapeDtypeStruct((1, 1), jnp.float32),
            jax.ShapeDtypeStruct((_ROWS, _DIM), jnp.float32),
            jax.ShapeDtypeStruct((1, 1), jnp.float32),
            jax.ShapeDtypeStruct((_GRID, 1, _TILE), jnp.int32),
        ],
        scratch_shapes=[
            pltpu.SMEM((1, 1), jnp.float32),
            pltpu.VMEM((1, _NUM_EMB), jnp.float32),
            pltpu.VMEM((1, _NUM_EMB), jnp.float32),
        ],
        compiler_params=pltpu.CompilerParams(
            dimension_semantics=("arbitrary",)),
        interpret=interpret,
    )(flat_x, embedding)
    quantized_st = qst.reshape(x.shape)
    encoding_indices = idx.reshape(x.shape[:1] + x.shape[2:])
    return (loss.reshape(()), quantized_st, perp.reshape(()), encoding_indices)
